# Initial kernel scaffold; baseline (speedup 1.0000x reference)
import jax
import jax.numpy as jnp
from jax.experimental import pallas as pl


def kernel(x, edge_index, W1, b1, W2, b2, cluster):
    raise NotImplementedError("write your pallas kernel here")


# speedup vs baseline: 12.9805x; 12.9805x over previous
import functools

import jax
import jax.numpy as jnp
from jax import lax
from jax.experimental import pallas as pl
from jax.experimental.pallas import tpu as pltpu
from jax.experimental.pallas import tpu_sc as plsc

N = 10000
NP = 10240
E = 320000
NTILES = 32
EPT = E // NTILES
B = 80
NCHUNK = EPT // B
RPT = NP // 16
ROWBLK = 1024
GRID = NP // ROWBLK


def _sc_mesh():
    return plsc.VectorSubcoreMesh(core_axis_name="c", subcore_axis_name="s",
                                  num_cores=2, num_subcores=16)


def _sc_degree(dst):
    @functools.partial(
        pl.kernel,
        out_type=jax.ShapeDtypeStruct((NTILES, NP), jnp.float32),
        mesh=_sc_mesh(),
        compiler_params=pltpu.CompilerParams(needs_layout_passes=False),
        scratch_types=[
            pltpu.VMEM((NP,), jnp.float32),
            pltpu.VMEM((B,), jnp.int32),
        ],
    )
    def k(dst_hbm, out_hbm, deg_v, idx_d):
        c = lax.axis_index("c")
        s = lax.axis_index("s")
        wid = c * 16 + s
        zeros = jnp.zeros((16,), jnp.float32)

        def zbody(j, carry):
            deg_v[pl.ds(j * 16, 16)] = zeros
            return carry

        lax.fori_loop(0, NP // 16, zbody, 0)
        ebase = wid * EPT
        ones = jnp.ones((16,), jnp.float32)

        def body(j, carry):
            b = pl.multiple_of(ebase + j * B, 8)
            pltpu.sync_copy(dst_hbm.at[pl.ds(b, B)], idx_d)
            for u in range(B // 16):
                idx16 = idx_d[pl.ds(u * 16, 16)]
                plsc.addupdate_scatter(deg_v, [idx16], ones)
            return carry

        lax.fori_loop(0, NCHUNK, body, 0)
        pltpu.sync_copy(deg_v, out_hbm.at[wid])

    return k(dst)


def _sc_pass(d, xwp, src, dst, zeros_blk):
    @functools.partial(
        pl.kernel,
        out_type=jax.ShapeDtypeStruct((2, NP, d), jnp.float32),
        mesh=_sc_mesh(),
        scratch_types=[
            pltpu.VMEM_SHARED((NP, d), jnp.float32),
            pltpu.VMEM((B,), jnp.int32),
            pltpu.VMEM((B,), jnp.int32),
            pltpu.VMEM((B, d), jnp.float32),
            pltpu.SemaphoreType.DMA,
        ],
    )
    def k(xwp_hbm, src_hbm, dst_hbm, zeros_hbm, out_hbm,
          acc_sh, idx_s, idx_d, rows_v, sem):
        c = lax.axis_index("c")
        s = lax.axis_index("s")
        wid = c * 16 + s
        r0 = s * RPT

        @pl.when(c == 0)
        def _():
            pltpu.sync_copy(xwp_hbm.at[pl.ds(r0, RPT), :],
                            acc_sh.at[pl.ds(r0, RPT), :])

        @pl.when(c != 0)
        def _():
            pltpu.sync_copy(zeros_hbm, acc_sh.at[pl.ds(r0, RPT), :])

        plsc.subcore_barrier()
        ebase = wid * EPT

        def body(j, carry):
            b = pl.multiple_of(ebase + j * B, 8)
            pltpu.sync_copy(src_hbm.at[pl.ds(b, B)], idx_s)
            pltpu.sync_copy(dst_hbm.at[pl.ds(b, B)], idx_d)
            pltpu.async_copy(xwp_hbm.at[idx_s], rows_v, sem).wait()
            pltpu.sync_copy(rows_v, acc_sh.at[idx_d], add=True)
            return carry

        lax.fori_loop(0, NCHUNK, body, 0)
        plsc.subcore_barrier()
        pltpu.sync_copy(acc_sh.at[pl.ds(r0, RPT), :],
                        out_hbm.at[c, pl.ds(r0, RPT), :])

    return k(xwp, src, dst, zeros_blk)


def _tc_mm1(x, W1, degt):
    def body(x_ref, w_ref, deg_ref, xw_ref, dinv_ref):
        deg = jnp.sum(deg_ref[...], axis=1, keepdims=True) + 1.0
        dinv = lax.rsqrt(deg)
        dinv_ref[...] = jnp.broadcast_to(dinv, (ROWBLK, 16))
        xw = jnp.dot(x_ref[...], w_ref[...],
                     preferred_element_type=jnp.float32)
        xw_ref[...] = xw * dinv

    return pl.pallas_call(
        body,
        grid=(GRID,),
        in_specs=[
            pl.BlockSpec((ROWBLK, 128), lambda i: (i, 0)),
            pl.BlockSpec((128, 128), lambda i: (0, 0)),
            pl.BlockSpec((ROWBLK, NTILES), lambda i: (i, 0)),
        ],
        out_specs=[
            pl.BlockSpec((ROWBLK, 128), lambda i: (i, 0)),
            pl.BlockSpec((ROWBLK, 16), lambda i: (i, 0)),
        ],
        out_shape=[
            jax.ShapeDtypeStruct((NP, 128), jnp.float32),
            jax.ShapeDtypeStruct((NP, 16), jnp.float32),
        ],
    )(x, W1, degt)


def _tc_mm2(acc1, dinvw, b1, W2):
    def body(a_ref, dinv_ref, b_ref, w_ref, out_ref):
        dinv = dinv_ref[...][:, 0:1]
        pre = (a_ref[0] + a_ref[1]) * dinv + b_ref[...]
        h = jnp.maximum(pre, 0.0)
        xw = jnp.dot(h, w_ref[...], preferred_element_type=jnp.float32)
        out_ref[...] = xw * dinv

    return pl.pallas_call(
        body,
        grid=(GRID,),
        in_specs=[
            pl.BlockSpec((2, ROWBLK, 128), lambda i: (0, i, 0)),
            pl.BlockSpec((ROWBLK, 16), lambda i: (i, 0)),
            pl.BlockSpec((1, 128), lambda i: (0, 0)),
            pl.BlockSpec((128, 128), lambda i: (0, 0)),
        ],
        out_specs=pl.BlockSpec((ROWBLK, 128), lambda i: (i, 0)),
        out_shape=jax.ShapeDtypeStruct((NP, 128), jnp.float32),
    )(acc1, dinvw, b1, W2)


def _tc_final(acc2, dinvw, b2, clusterT):
    def body(a_ref, dinv_ref, b_ref, ct_ref, feat_ref, q_ref):
        dinv = dinv_ref[...][:, 0:1]
        f = (a_ref[0] + a_ref[1])[:, :64] * dinv + b_ref[...]
        feat_ref[...] = f
        ct = ct_ref[...]
        fsq = jnp.sum(f * f, axis=1, keepdims=True)
        csq = jnp.sum(ct * ct, axis=0, keepdims=True)
        fc = jnp.dot(f, ct, preferred_element_type=jnp.float32)
        dist = fsq - 2.0 * fc + csq
        u = 1.0 + dist * 0.5
        r = lax.rsqrt(u)
        qun = r * r * r
        q_ref[...] = qun / jnp.sum(qun, axis=1, keepdims=True)

    return pl.pallas_call(
        body,
        grid=(GRID,),
        in_specs=[
            pl.BlockSpec((2, ROWBLK, 128), lambda i: (0, i, 0)),
            pl.BlockSpec((ROWBLK, 16), lambda i: (i, 0)),
            pl.BlockSpec((1, 64), lambda i: (0, 0)),
            pl.BlockSpec((64, 16), lambda i: (0, 0)),
        ],
        out_specs=[
            pl.BlockSpec((ROWBLK, 64), lambda i: (i, 0)),
            pl.BlockSpec((ROWBLK, 16), lambda i: (i, 0)),
        ],
        out_shape=[
            jax.ShapeDtypeStruct((NP, 64), jnp.float32),
            jax.ShapeDtypeStruct((NP, 16), jnp.float32),
        ],
    )(acc2, dinvw, b2, clusterT)


def kernel(x, edge_index, W1, b1, W2, b2, cluster):
    src = edge_index[0].astype(jnp.int32)
    dst = edge_index[1].astype(jnp.int32)
    x_p = jnp.zeros((NP, 128), jnp.float32).at[:N].set(x)

    zeros128 = jnp.zeros((RPT, 128), jnp.float32)

    W2p = jnp.pad(W2, ((0, 0), (0, 64)))
    degt = _sc_degree(dst).T
    xw1p, dinvw = _tc_mm1(x_p, W1, degt)
    acc1 = _sc_pass(128, xw1p, src, dst, zeros128)
    xw2p = _tc_mm2(acc1, dinvw, b1.reshape(1, 128), W2p)
    acc2 = _sc_pass(128, xw2p, src, dst, zeros128)
    feature, q = _tc_final(acc2, dinvw, b2.reshape(1, 64), cluster.T)
    return (feature[:N], q[:N])


# speedup vs baseline: 24.2478x; 1.8680x over previous
import functools

import jax
import jax.numpy as jnp
from jax import lax
from jax.experimental import pallas as pl
from jax.experimental.pallas import tpu as pltpu
from jax.experimental.pallas import tpu_sc as plsc

N = 10000
NP = 10240
E = 320000
NTILES = 32
EPT = E // NTILES
B = 80
NCHUNK = EPT // B
PCHUNK = 25
NPHASE = NCHUNK // PCHUNK
RPT = NP // 16
ROWBLK = 1024
GRID = NP // ROWBLK


def _sc_mesh():
    return plsc.VectorSubcoreMesh(core_axis_name="c", subcore_axis_name="s",
                                  num_cores=2, num_subcores=16)


def _sc_degree(dst):
    @functools.partial(
        pl.kernel,
        out_type=jax.ShapeDtypeStruct((NTILES, NP), jnp.float32),
        mesh=_sc_mesh(),
        compiler_params=pltpu.CompilerParams(needs_layout_passes=False),
        scratch_types=[
            pltpu.VMEM((NP,), jnp.float32),
            pltpu.VMEM((EPT,), jnp.int32),
        ],
    )
    def k(dst_hbm, out_hbm, deg_v, idx_d):
        c = lax.axis_index("c")
        s = lax.axis_index("s")
        wid = c * 16 + s
        ebase = pl.multiple_of(wid * EPT, 8)
        pltpu.sync_copy(dst_hbm.at[pl.ds(ebase, EPT)], idx_d)
        zeros = jnp.zeros((16,), jnp.float32)

        def zbody(j, carry):
            deg_v[pl.ds(j * 16, 16)] = zeros
            return carry

        lax.fori_loop(0, NP // 16, zbody, 0)
        ones = jnp.ones((16,), jnp.float32)

        def body(j, carry):
            idx16 = idx_d[pl.ds(j * 16, 16)]
            plsc.addupdate_scatter(deg_v, [idx16], ones)
            return carry

        lax.fori_loop(0, EPT // 16, body, 0, unroll=8)
        pltpu.sync_copy(deg_v, out_hbm.at[wid])

    return k(dst)


def _sc_pass(d, xwp, src3, dst3, zeros_blk):
    @functools.partial(
        pl.kernel,
        out_type=jax.ShapeDtypeStruct((2, NP, d), jnp.float32),
        mesh=_sc_mesh(),
        scratch_types=[
            pltpu.VMEM_SHARED((NP, d), jnp.float32),
            pltpu.VMEM((2, PCHUNK, B), jnp.int32),
            pltpu.VMEM((2, PCHUNK, B), jnp.int32),
            pltpu.VMEM((2, B, d), jnp.float32),
            pltpu.SemaphoreType.DMA,
            pltpu.SemaphoreType.DMA,
        ],
    )
    def k(xwp_hbm, src_hbm, dst_hbm, zeros_hbm, out_hbm,
          acc_sh, src_v, dst_v, rows_v, sem, sem_i):
        c = lax.axis_index("c")
        s = lax.axis_index("s")
        wid = c * 16 + s
        r0 = s * RPT

        pltpu.async_copy(src_hbm.at[wid * NPHASE], src_v.at[0], sem_i)
        pltpu.async_copy(dst_hbm.at[wid * NPHASE], dst_v.at[0], sem_i)

        @pl.when(c == 0)
        def _():
            pltpu.sync_copy(xwp_hbm.at[pl.ds(r0, RPT), :],
                            acc_sh.at[pl.ds(r0, RPT), :])

        @pl.when(c != 0)
        def _():
            pltpu.sync_copy(zeros_hbm, acc_sh.at[pl.ds(r0, RPT), :])

        plsc.subcore_barrier()

        for p in range(NPHASE):
            pb = p % 2
            pltpu.make_async_copy(src_hbm.at[wid * NPHASE + p],
                                  src_v.at[pb], sem_i).wait()
            pltpu.make_async_copy(dst_hbm.at[wid * NPHASE + p],
                                  dst_v.at[pb], sem_i).wait()
            if p + 1 < NPHASE:
                pltpu.async_copy(src_hbm.at[wid * NPHASE + p + 1],
                                 src_v.at[1 - pb], sem_i)
                pltpu.async_copy(dst_hbm.at[wid * NPHASE + p + 1],
                                 dst_v.at[1 - pb], sem_i)

            pltpu.async_copy(xwp_hbm.at[src_v.at[pb, 0]], rows_v.at[0], sem)

            def body(j, carry):
                buf = lax.rem(j, 2)
                pltpu.make_async_copy(xwp_hbm.at[src_v.at[pb, j]],
                                      rows_v.at[buf], sem).wait()

                @pl.when(j + 1 < PCHUNK)
                def _():
                    pltpu.async_copy(xwp_hbm.at[src_v.at[pb, j + 1]],
                                     rows_v.at[1 - buf], sem)

                pltpu.sync_copy(rows_v.at[buf], acc_sh.at[dst_v.at[pb, j]],
                                add=True)
                return carry

            lax.fori_loop(0, PCHUNK, body, 0)

        plsc.subcore_barrier()
        pltpu.sync_copy(acc_sh.at[pl.ds(r0, RPT), :],
                        out_hbm.at[c, pl.ds(r0, RPT), :])

    return k(xwp, src3, dst3, zeros_blk)


def _tc_mm1(x, W1, degt):
    def body(x_ref, w_ref, deg_ref, xw_ref, dinv_ref):
        deg = jnp.sum(deg_ref[...], axis=1, keepdims=True) + 1.0
        dinv = lax.rsqrt(deg)
        dinv_ref[...] = jnp.broadcast_to(dinv, (ROWBLK, 16))
        xw = jnp.dot(x_ref[...], w_ref[...],
                     preferred_element_type=jnp.float32)
        xw_ref[...] = xw * dinv

    return pl.pallas_call(
        body,
        grid=(GRID,),
        in_specs=[
            pl.BlockSpec((ROWBLK, 128), lambda i: (i, 0)),
            pl.BlockSpec((128, 128), lambda i: (0, 0)),
            pl.BlockSpec((ROWBLK, NTILES), lambda i: (i, 0)),
        ],
        out_specs=[
            pl.BlockSpec((ROWBLK, 128), lambda i: (i, 0)),
            pl.BlockSpec((ROWBLK, 16), lambda i: (i, 0)),
        ],
        out_shape=[
            jax.ShapeDtypeStruct((NP, 128), jnp.float32),
            jax.ShapeDtypeStruct((NP, 16), jnp.float32),
        ],
    )(x, W1, degt)


def _tc_mm2(acc1, dinvw, b1, W2):
    def body(a_ref, dinv_ref, b_ref, w_ref, out_ref):
        dinv = dinv_ref[...][:, 0:1]
        pre = (a_ref[0] + a_ref[1]) * dinv + b_ref[...]
        h = jnp.maximum(pre, 0.0)
        xw = jnp.dot(h, w_ref[...], preferred_element_type=jnp.float32)
        out_ref[...] = xw * dinv

    return pl.pallas_call(
        body,
        grid=(GRID,),
        in_specs=[
            pl.BlockSpec((2, ROWBLK, 128), lambda i: (0, i, 0)),
            pl.BlockSpec((ROWBLK, 16), lambda i: (i, 0)),
            pl.BlockSpec((1, 128), lambda i: (0, 0)),
            pl.BlockSpec((128, 128), lambda i: (0, 0)),
        ],
        out_specs=pl.BlockSpec((ROWBLK, 128), lambda i: (i, 0)),
        out_shape=jax.ShapeDtypeStruct((NP, 128), jnp.float32),
    )(acc1, dinvw, b1, W2)


def _tc_final(acc2, dinvw, b2, clusterT):
    def body(a_ref, dinv_ref, b_ref, ct_ref, feat_ref, q_ref):
        dinv = dinv_ref[...][:, 0:1]
        f = (a_ref[0] + a_ref[1])[:, :64] * dinv + b_ref[...]
        feat_ref[...] = f
        ct = ct_ref[...]
        fsq = jnp.sum(f * f, axis=1, keepdims=True)
        csq = jnp.sum(ct * ct, axis=0, keepdims=True)
        fc = jnp.dot(f, ct, preferred_element_type=jnp.float32)
        dist = fsq - 2.0 * fc + csq
        u = 1.0 + dist * 0.5
        r = lax.rsqrt(u)
        qun = r * r * r
        q_ref[...] = qun / jnp.sum(qun, axis=1, keepdims=True)

    return pl.pallas_call(
        body,
        grid=(GRID,),
        in_specs=[
            pl.BlockSpec((2, ROWBLK, 128), lambda i: (0, i, 0)),
            pl.BlockSpec((ROWBLK, 16), lambda i: (i, 0)),
            pl.BlockSpec((1, 64), lambda i: (0, 0)),
            pl.BlockSpec((64, 16), lambda i: (0, 0)),
        ],
        out_specs=[
            pl.BlockSpec((ROWBLK, 64), lambda i: (i, 0)),
            pl.BlockSpec((ROWBLK, 16), lambda i: (i, 0)),
        ],
        out_shape=[
            jax.ShapeDtypeStruct((NP, 64), jnp.float32),
            jax.ShapeDtypeStruct((NP, 16), jnp.float32),
        ],
    )(acc2, dinvw, b2, clusterT)


def kernel(x, edge_index, W1, b1, W2, b2, cluster):
    src = edge_index[0].astype(jnp.int32)
    dst = edge_index[1].astype(jnp.int32)
    x_p = jnp.zeros((NP, 128), jnp.float32).at[:N].set(x)

    zeros128 = jnp.zeros((RPT, 128), jnp.float32)

    W2p = jnp.pad(W2, ((0, 0), (0, 64)))
    src3 = src.reshape(NTILES * NPHASE, PCHUNK, B)
    dst3 = dst.reshape(NTILES * NPHASE, PCHUNK, B)
    degt = _sc_degree(dst).T
    xw1p, dinvw = _tc_mm1(x_p, W1, degt)
    acc1 = _sc_pass(128, xw1p, src3, dst3, zeros128)
    xw2p = _tc_mm2(acc1, dinvw, b1.reshape(1, 128), W2p)
    acc2 = _sc_pass(128, xw2p, src3, dst3, zeros128)
    feature, q = _tc_final(acc2, dinvw, b2.reshape(1, 64), cluster.T)
    return (feature[:N], q[:N])


# speedup vs baseline: 26.0704x; 1.0752x over previous
import functools

import jax
import jax.numpy as jnp
from jax import lax
from jax.experimental import pallas as pl
from jax.experimental.pallas import tpu as pltpu
from jax.experimental.pallas import tpu_sc as plsc

N = 10000
NP = 10240
E = 320000
NTILES = 32
EPT = E // NTILES
B = 80
NCHUNK = EPT // B
PCHUNK = 25
NPHASE = NCHUNK // PCHUNK
RPT = NP // 16
ROWBLK = 1024
GRID = NP // ROWBLK


def _sc_mesh():
    return plsc.VectorSubcoreMesh(core_axis_name="c", subcore_axis_name="s",
                                  num_cores=2, num_subcores=16)


def _sc_degree(dst):
    @functools.partial(
        pl.kernel,
        out_type=jax.ShapeDtypeStruct((NTILES, NP), jnp.float32),
        mesh=_sc_mesh(),
        compiler_params=pltpu.CompilerParams(needs_layout_passes=False),
        scratch_types=[
            pltpu.VMEM((NP,), jnp.float32),
            pltpu.VMEM((EPT,), jnp.int32),
        ],
    )
    def k(dst_hbm, out_hbm, deg_v, idx_d):
        c = lax.axis_index("c")
        s = lax.axis_index("s")
        wid = c * 16 + s
        ebase = pl.multiple_of(wid * EPT, 8)
        pltpu.sync_copy(dst_hbm.at[pl.ds(ebase, EPT)], idx_d)
        zeros = jnp.zeros((16,), jnp.float32)

        def zbody(j, carry):
            deg_v[pl.ds(j * 16, 16)] = zeros
            return carry

        lax.fori_loop(0, NP // 16, zbody, 0)
        ones = jnp.ones((16,), jnp.float32)

        def body(j, carry):
            idx16 = idx_d[pl.ds(j * 16, 16)]
            plsc.addupdate_scatter(deg_v, [idx16], ones)
            return carry

        lax.fori_loop(0, EPT // 16, body, 0, unroll=8)
        pltpu.sync_copy(deg_v, out_hbm.at[wid])

    return k(dst)


def _sc_pass(d, xwp, src3, dst3, zeros_blk):
    @functools.partial(
        pl.kernel,
        out_type=jax.ShapeDtypeStruct((2, NP, d), jnp.float32),
        mesh=_sc_mesh(),
        compiler_params=pltpu.CompilerParams(
            use_tc_tiling_on_sc=(d % 128 == 0)),
        scratch_types=[
            pltpu.VMEM_SHARED((NP, d), jnp.float32),
            pltpu.VMEM((2, PCHUNK, B), jnp.int32),
            pltpu.VMEM((2, PCHUNK, B), jnp.int32),
            pltpu.VMEM((2, B, d), jnp.float32),
            pltpu.SemaphoreType.DMA,
            pltpu.SemaphoreType.DMA,
        ],
    )
    def k(xwp_hbm, src_hbm, dst_hbm, zeros_hbm, out_hbm,
          acc_sh, src_v, dst_v, rows_v, sem, sem_i):
        c = lax.axis_index("c")
        s = lax.axis_index("s")
        wid = c * 16 + s
        r0 = s * RPT

        pltpu.async_copy(src_hbm.at[wid * NPHASE], src_v.at[0], sem_i)
        pltpu.async_copy(dst_hbm.at[wid * NPHASE], dst_v.at[0], sem_i)

        @pl.when(c == 0)
        def _():
            pltpu.sync_copy(xwp_hbm.at[pl.ds(r0, RPT), :],
                            acc_sh.at[pl.ds(r0, RPT), :])

        @pl.when(c != 0)
        def _():
            pltpu.sync_copy(zeros_hbm, acc_sh.at[pl.ds(r0, RPT), :])

        plsc.subcore_barrier()

        for p in range(NPHASE):
            pb = p % 2
            pltpu.make_async_copy(src_hbm.at[wid * NPHASE + p],
                                  src_v.at[pb], sem_i).wait()
            pltpu.make_async_copy(dst_hbm.at[wid * NPHASE + p],
                                  dst_v.at[pb], sem_i).wait()
            if p + 1 < NPHASE:
                pltpu.async_copy(src_hbm.at[wid * NPHASE + p + 1],
                                 src_v.at[1 - pb], sem_i)
                pltpu.async_copy(dst_hbm.at[wid * NPHASE + p + 1],
                                 dst_v.at[1 - pb], sem_i)

            pltpu.async_copy(xwp_hbm.at[src_v.at[pb, 0]], rows_v.at[0], sem)

            def body(j, carry):
                buf = lax.rem(j, 2)
                pltpu.make_async_copy(xwp_hbm.at[src_v.at[pb, j]],
                                      rows_v.at[buf], sem).wait()

                @pl.when(j + 1 < PCHUNK)
                def _():
                    pltpu.async_copy(xwp_hbm.at[src_v.at[pb, j + 1]],
                                     rows_v.at[1 - buf], sem)

                pltpu.sync_copy(rows_v.at[buf], acc_sh.at[dst_v.at[pb, j]],
                                add=True)
                return carry

            lax.fori_loop(0, PCHUNK, body, 0)

        plsc.subcore_barrier()
        pltpu.sync_copy(acc_sh.at[pl.ds(r0, RPT), :],
                        out_hbm.at[c, pl.ds(r0, RPT), :])

    return k(xwp, src3, dst3, zeros_blk)


def _tc_mm1(x, W1, degt):
    def body(x_ref, w_ref, deg_ref, xw_ref, dinv_ref):
        deg = jnp.sum(deg_ref[...], axis=1, keepdims=True) + 1.0
        dinv = lax.rsqrt(deg)
        dinv_ref[...] = jnp.broadcast_to(dinv, (ROWBLK, 16))
        xw = jnp.dot(x_ref[...], w_ref[...],
                     preferred_element_type=jnp.float32)
        xw_ref[...] = xw * dinv

    return pl.pallas_call(
        body,
        grid=(GRID,),
        in_specs=[
            pl.BlockSpec((ROWBLK, 128), lambda i: (i, 0)),
            pl.BlockSpec((128, 128), lambda i: (0, 0)),
            pl.BlockSpec((ROWBLK, NTILES), lambda i: (i, 0)),
        ],
        out_specs=[
            pl.BlockSpec((ROWBLK, 128), lambda i: (i, 0)),
            pl.BlockSpec((ROWBLK, 16), lambda i: (i, 0)),
        ],
        out_shape=[
            jax.ShapeDtypeStruct((NP, 128), jnp.float32),
            jax.ShapeDtypeStruct((NP, 16), jnp.float32),
        ],
    )(x, W1, degt)


def _tc_mm2(acc1, dinvw, b1, W2):
    def body(a_ref, dinv_ref, b_ref, w_ref, out_ref):
        dinv = dinv_ref[...][:, 0:1]
        pre = (a_ref[0] + a_ref[1]) * dinv + b_ref[...]
        h = jnp.maximum(pre, 0.0)
        xw = jnp.dot(h, w_ref[...], preferred_element_type=jnp.float32)
        out_ref[...] = xw * dinv

    return pl.pallas_call(
        body,
        grid=(GRID,),
        in_specs=[
            pl.BlockSpec((2, ROWBLK, 128), lambda i: (0, i, 0)),
            pl.BlockSpec((ROWBLK, 16), lambda i: (i, 0)),
            pl.BlockSpec((1, 128), lambda i: (0, 0)),
            pl.BlockSpec((128, 64), lambda i: (0, 0)),
        ],
        out_specs=pl.BlockSpec((ROWBLK, 64), lambda i: (i, 0)),
        out_shape=jax.ShapeDtypeStruct((NP, 64), jnp.float32),
    )(acc1, dinvw, b1, W2)


def _tc_final(acc2, dinvw, b2, clusterT):
    def body(a_ref, dinv_ref, b_ref, ct_ref, feat_ref, q_ref):
        dinv = dinv_ref[...][:, 0:1]
        f = (a_ref[0] + a_ref[1]) * dinv + b_ref[...]
        feat_ref[...] = f
        ct = ct_ref[...]
        fsq = jnp.sum(f * f, axis=1, keepdims=True)
        csq = jnp.sum(ct * ct, axis=0, keepdims=True)
        fc = jnp.dot(f, ct, preferred_element_type=jnp.float32)
        dist = fsq - 2.0 * fc + csq
        u = 1.0 + dist * 0.5
        r = lax.rsqrt(u)
        qun = r * r * r
        q_ref[...] = qun / jnp.sum(qun, axis=1, keepdims=True)

    return pl.pallas_call(
        body,
        grid=(GRID,),
        in_specs=[
            pl.BlockSpec((2, ROWBLK, 64), lambda i: (0, i, 0)),
            pl.BlockSpec((ROWBLK, 16), lambda i: (i, 0)),
            pl.BlockSpec((1, 64), lambda i: (0, 0)),
            pl.BlockSpec((64, 16), lambda i: (0, 0)),
        ],
        out_specs=[
            pl.BlockSpec((ROWBLK, 64), lambda i: (i, 0)),
            pl.BlockSpec((ROWBLK, 16), lambda i: (i, 0)),
        ],
        out_shape=[
            jax.ShapeDtypeStruct((NP, 64), jnp.float32),
            jax.ShapeDtypeStruct((NP, 16), jnp.float32),
        ],
    )(acc2, dinvw, b2, clusterT)


def kernel(x, edge_index, W1, b1, W2, b2, cluster):
    src = edge_index[0].astype(jnp.int32)
    dst = edge_index[1].astype(jnp.int32)
    x_p = jnp.zeros((NP, 128), jnp.float32).at[:N].set(x)

    zeros128 = jnp.zeros((RPT, 128), jnp.float32)
    zeros64 = jnp.zeros((RPT, 64), jnp.float32)

    src3 = src.reshape(NTILES * NPHASE, PCHUNK, B)
    dst3 = dst.reshape(NTILES * NPHASE, PCHUNK, B)
    degt = _sc_degree(dst).T
    xw1p, dinvw = _tc_mm1(x_p, W1, degt)
    acc1 = _sc_pass(128, xw1p, src3, dst3, zeros128)
    xw2p = _tc_mm2(acc1, dinvw, b1.reshape(1, 128), W2)
    acc2 = _sc_pass(64, xw2p, src3, dst3, zeros64)
    feature, q = _tc_final(acc2, dinvw, b2.reshape(1, 64), cluster.T)
    return (feature[:N], q[:N])


# speedup vs baseline: 28.6386x; 1.0985x over previous
import functools

import jax
import jax.numpy as jnp
from jax import lax
from jax.experimental import pallas as pl
from jax.experimental.pallas import tpu as pltpu
from jax.experimental.pallas import tpu_sc as plsc

N = 10000
NP = 10240
E = 320000
NTILES = 32
EPT = E // NTILES
B1, PC1, NPH1 = 100, 20, 5
B2, PC2, NPH2 = 125, 16, 5
BD = 80
RPT = NP // 16
ROWBLK = 1024
GRID = NP // ROWBLK


def _sc_mesh():
    return plsc.VectorSubcoreMesh(core_axis_name="c", subcore_axis_name="s",
                                  num_cores=2, num_subcores=16)


def _sc_degree(dst):
    @functools.partial(
        pl.kernel,
        out_type=jax.ShapeDtypeStruct((NTILES, NP), jnp.float32),
        mesh=_sc_mesh(),
        compiler_params=pltpu.CompilerParams(needs_layout_passes=False),
        scratch_types=[
            pltpu.VMEM((NP,), jnp.float32),
            pltpu.VMEM((EPT,), jnp.int32),
        ],
    )
    def k(dst_hbm, out_hbm, deg_v, idx_d):
        c = lax.axis_index("c")
        s = lax.axis_index("s")
        wid = c * 16 + s
        ebase = pl.multiple_of(wid * EPT, 8)
        pltpu.sync_copy(dst_hbm.at[pl.ds(ebase, EPT)], idx_d)
        zeros = jnp.zeros((16,), jnp.float32)

        def zbody(j, carry):
            deg_v[pl.ds(j * 16, 16)] = zeros
            return carry

        lax.fori_loop(0, NP // 16, zbody, 0)
        ones = jnp.ones((16,), jnp.float32)

        def body(j, carry):
            idx16 = idx_d[pl.ds(j * 16, 16)]
            plsc.addupdate_scatter(deg_v, [idx16], ones)
            return carry

        lax.fori_loop(0, EPT // 16, body, 0, unroll=8)
        pltpu.sync_copy(deg_v, out_hbm.at[wid])

    return k(dst)


def _sc_pass(d, bb, pc, nph, xwp, src3, dst3, zeros_blk):
    @functools.partial(
        pl.kernel,
        out_type=jax.ShapeDtypeStruct((2, NP, d), jnp.float32),
        mesh=_sc_mesh(),
        compiler_params=pltpu.CompilerParams(
            use_tc_tiling_on_sc=(d % 128 == 0)),
        scratch_types=[
            pltpu.VMEM_SHARED((NP, d), jnp.float32),
            pltpu.VMEM((2, pc, bb), jnp.int32),
            pltpu.VMEM((2, pc, bb), jnp.int32),
            pltpu.VMEM((2, bb, d), jnp.float32),
            pltpu.SemaphoreType.DMA,
            pltpu.SemaphoreType.DMA,
            pltpu.SemaphoreType.DMA,
            pltpu.SemaphoreType.DMA,
        ],
    )
    def k(xwp_hbm, src_hbm, dst_hbm, zeros_hbm, out_hbm,
          acc_sh, src_v, dst_v, rows_v, semg, sems0, sems1, sem_i):
        c = lax.axis_index("c")
        s = lax.axis_index("s")
        wid = c * 16 + s
        r0 = s * RPT

        pltpu.async_copy(src_hbm.at[wid * nph], src_v.at[0], sem_i)
        pltpu.async_copy(dst_hbm.at[wid * nph], dst_v.at[0], sem_i)

        @pl.when(c == 0)
        def _():
            pltpu.sync_copy(xwp_hbm.at[pl.ds(r0, RPT), :],
                            acc_sh.at[pl.ds(r0, RPT), :])

        @pl.when(c != 0)
        def _():
            pltpu.sync_copy(zeros_hbm, acc_sh.at[pl.ds(r0, RPT), :])

        plsc.subcore_barrier()

        for p in range(nph):
            pb = p % 2
            pltpu.make_async_copy(src_hbm.at[wid * nph + p],
                                  src_v.at[pb], sem_i).wait()
            pltpu.make_async_copy(dst_hbm.at[wid * nph + p],
                                  dst_v.at[pb], sem_i).wait()
            if p + 1 < nph:
                pltpu.async_copy(src_hbm.at[wid * nph + p + 1],
                                 src_v.at[1 - pb], sem_i)
                pltpu.async_copy(dst_hbm.at[wid * nph + p + 1],
                                 dst_v.at[1 - pb], sem_i)

            pltpu.async_copy(xwp_hbm.at[src_v.at[pb, 0]], rows_v.at[0], semg)

            def body(j2, carry):
                j0 = j2 * 2
                pltpu.make_async_copy(xwp_hbm.at[src_v.at[pb, j0]],
                                      rows_v.at[0], semg).wait()
                pltpu.async_copy(rows_v.at[0], acc_sh.at[dst_v.at[pb, j0]],
                                 sems0, add=True)

                @pl.when(j2 > 0)
                def _():
                    pltpu.make_async_copy(
                        rows_v.at[1], acc_sh.at[dst_v.at[pb, j0 - 1]],
                        sems1).wait()

                pltpu.async_copy(xwp_hbm.at[src_v.at[pb, j0 + 1]],
                                 rows_v.at[1], semg)
                pltpu.make_async_copy(xwp_hbm.at[src_v.at[pb, j0 + 1]],
                                      rows_v.at[1], semg).wait()
                pltpu.async_copy(rows_v.at[1],
                                 acc_sh.at[dst_v.at[pb, j0 + 1]],
                                 sems1, add=True)
                pltpu.make_async_copy(rows_v.at[0],
                                      acc_sh.at[dst_v.at[pb, j0]],
                                      sems0).wait()

                @pl.when(j0 + 2 < pc)
                def _():
                    pltpu.async_copy(xwp_hbm.at[src_v.at[pb, j0 + 2]],
                                     rows_v.at[0], semg)

                return carry

            lax.fori_loop(0, pc // 2, body, 0)
            pltpu.make_async_copy(rows_v.at[1],
                                  acc_sh.at[dst_v.at[pb, pc - 1]],
                                  sems1).wait()

        plsc.subcore_barrier()
        pltpu.sync_copy(acc_sh.at[pl.ds(r0, RPT), :],
                        out_hbm.at[c, pl.ds(r0, RPT), :])

    return k(xwp, src3, dst3, zeros_blk)


def _tc_mm1(x, W1, degt):
    def body(x_ref, w_ref, deg_ref, xw_ref, dinv_ref):
        deg = jnp.sum(deg_ref[...], axis=1, keepdims=True) + 1.0
        dinv = lax.rsqrt(deg)
        dinv_ref[...] = jnp.broadcast_to(dinv, (ROWBLK, 16))
        xw = jnp.dot(x_ref[...], w_ref[...],
                     preferred_element_type=jnp.float32)
        xw_ref[...] = xw * dinv

    return pl.pallas_call(
        body,
        grid=(GRID,),
        in_specs=[
            pl.BlockSpec((ROWBLK, 128), lambda i: (i, 0)),
            pl.BlockSpec((128, 128), lambda i: (0, 0)),
            pl.BlockSpec((ROWBLK, NTILES), lambda i: (i, 0)),
        ],
        out_specs=[
            pl.BlockSpec((ROWBLK, 128), lambda i: (i, 0)),
            pl.BlockSpec((ROWBLK, 16), lambda i: (i, 0)),
        ],
        out_shape=[
            jax.ShapeDtypeStruct((NP, 128), jnp.float32),
            jax.ShapeDtypeStruct((NP, 16), jnp.float32),
        ],
    )(x, W1, degt)


def _tc_mm2(acc1, dinvw, b1, W2):
    def body(a_ref, dinv_ref, b_ref, w_ref, out_ref):
        dinv = dinv_ref[...][:, 0:1]
        pre = (a_ref[0] + a_ref[1]) * dinv + b_ref[...]
        h = jnp.maximum(pre, 0.0)
        xw = jnp.dot(h, w_ref[...], preferred_element_type=jnp.float32)
        out_ref[...] = xw * dinv

    return pl.pallas_call(
        body,
        grid=(GRID,),
        in_specs=[
            pl.BlockSpec((2, ROWBLK, 128), lambda i: (0, i, 0)),
            pl.BlockSpec((ROWBLK, 16), lambda i: (i, 0)),
            pl.BlockSpec((1, 128), lambda i: (0, 0)),
            pl.BlockSpec((128, 64), lambda i: (0, 0)),
        ],
        out_specs=pl.BlockSpec((ROWBLK, 64), lambda i: (i, 0)),
        out_shape=jax.ShapeDtypeStruct((NP, 64), jnp.float32),
    )(acc1, dinvw, b1, W2)


def _tc_final(acc2, dinvw, b2, clusterT):
    def body(a_ref, dinv_ref, b_ref, ct_ref, feat_ref, q_ref):
        dinv = dinv_ref[...][:, 0:1]
        f = (a_ref[0] + a_ref[1]) * dinv + b_ref[...]
        feat_ref[...] = f
        ct = ct_ref[...]
        fsq = jnp.sum(f * f, axis=1, keepdims=True)
        csq = jnp.sum(ct * ct, axis=0, keepdims=True)
        fc = jnp.dot(f, ct, preferred_element_type=jnp.float32)
        dist = fsq - 2.0 * fc + csq
        u = 1.0 + dist * 0.5
        r = lax.rsqrt(u)
        qun = r * r * r
        q_ref[...] = qun / jnp.sum(qun, axis=1, keepdims=True)

    return pl.pallas_call(
        body,
        grid=(GRID,),
        in_specs=[
            pl.BlockSpec((2, ROWBLK, 64), lambda i: (0, i, 0)),
            pl.BlockSpec((ROWBLK, 16), lambda i: (i, 0)),
            pl.BlockSpec((1, 64), lambda i: (0, 0)),
            pl.BlockSpec((64, 16), lambda i: (0, 0)),
        ],
        out_specs=[
            pl.BlockSpec((ROWBLK, 64), lambda i: (i, 0)),
            pl.BlockSpec((ROWBLK, 16), lambda i: (i, 0)),
        ],
        out_shape=[
            jax.ShapeDtypeStruct((NP, 64), jnp.float32),
            jax.ShapeDtypeStruct((NP, 16), jnp.float32),
        ],
    )(acc2, dinvw, b2, clusterT)


def kernel(x, edge_index, W1, b1, W2, b2, cluster):
    src = edge_index[0].astype(jnp.int32)
    dst = edge_index[1].astype(jnp.int32)
    x_p = jnp.zeros((NP, 128), jnp.float32).at[:N].set(x)

    zeros128 = jnp.zeros((RPT, 128), jnp.float32)
    zeros64 = jnp.zeros((RPT, 64), jnp.float32)

    src3a = src.reshape(NTILES * NPH1, PC1, B1)
    dst3a = dst.reshape(NTILES * NPH1, PC1, B1)
    src3b = src.reshape(NTILES * NPH2, PC2, B2)
    dst3b = dst.reshape(NTILES * NPH2, PC2, B2)
    degt = _sc_degree(dst).T
    xw1p, dinvw = _tc_mm1(x_p, W1, degt)
    acc1 = _sc_pass(128, B1, PC1, NPH1, xw1p, src3a, dst3a, zeros128)
    xw2p = _tc_mm2(acc1, dinvw, b1.reshape(1, 128), W2)
    acc2 = _sc_pass(64, B2, PC2, NPH2, xw2p, src3b, dst3b, zeros64)
    feature, q = _tc_final(acc2, dinvw, b2.reshape(1, 64), cluster.T)
    return (feature[:N], q[:N])


# speedup vs baseline: 30.7656x; 1.0743x over previous
import functools

import jax
import jax.numpy as jnp
from jax import lax
from jax.experimental import pallas as pl
from jax.experimental.pallas import tpu as pltpu
from jax.experimental.pallas import tpu_sc as plsc

N = 10000
NP = 10240
E = 320000
NTILES = 32
EPT = E // NTILES
B1, PC1, NPH1, R1 = 50, 40, 5, 4
B2, PC2, NPH2, R2 = 200, 10, 5, 2
RPT = NP // 16
ROWBLK = 1024
GRID = NP // ROWBLK


def _sc_mesh():
    return plsc.VectorSubcoreMesh(core_axis_name="c", subcore_axis_name="s",
                                  num_cores=2, num_subcores=16)


def _sc_degree(dst):
    @functools.partial(
        pl.kernel,
        out_type=jax.ShapeDtypeStruct((NTILES, NP), jnp.float32),
        mesh=_sc_mesh(),
        compiler_params=pltpu.CompilerParams(needs_layout_passes=False),
        scratch_types=[
            pltpu.VMEM((NP,), jnp.float32),
            pltpu.VMEM((EPT,), jnp.int32),
        ],
    )
    def k(dst_hbm, out_hbm, deg_v, idx_d):
        c = lax.axis_index("c")
        s = lax.axis_index("s")
        wid = c * 16 + s
        ebase = pl.multiple_of(wid * EPT, 8)
        pltpu.sync_copy(dst_hbm.at[pl.ds(ebase, EPT)], idx_d)
        zeros = jnp.zeros((16,), jnp.float32)

        def zbody(j, carry):
            deg_v[pl.ds(j * 16, 16)] = zeros
            return carry

        lax.fori_loop(0, NP // 16, zbody, 0)
        ones = jnp.ones((16,), jnp.float32)

        def body(j, carry):
            idx16 = idx_d[pl.ds(j * 16, 16)]
            plsc.addupdate_scatter(deg_v, [idx16], ones)
            return carry

        lax.fori_loop(0, EPT // 16, body, 0, unroll=8)
        pltpu.sync_copy(deg_v, out_hbm.at[wid])

    return k(dst)


def _sc_pass(d, bb, pc, nph, ring, xwp, src3, dst3, zeros_blk):
    G = ring - 2 if ring > 2 else 1

    @functools.partial(
        pl.kernel,
        out_type=jax.ShapeDtypeStruct((2, NP, d), jnp.float32),
        mesh=_sc_mesh(),
        compiler_params=pltpu.CompilerParams(
            use_tc_tiling_on_sc=(d % 128 == 0)),
        scratch_types=[
            pltpu.VMEM_SHARED((NP, d), jnp.float32),
            pltpu.VMEM((2, pc, bb), jnp.int32),
            pltpu.VMEM((2, pc, bb), jnp.int32),
            pltpu.VMEM((ring, bb, d), jnp.float32),
            [pltpu.SemaphoreType.DMA] * ring,
            [pltpu.SemaphoreType.DMA] * ring,
            pltpu.SemaphoreType.DMA,
        ],
    )
    def k(xwp_hbm, src_hbm, dst_hbm, zeros_hbm, out_hbm,
          acc_sh, src_v, dst_v, rows_v, sg, ss, sem_i):
        c = lax.axis_index("c")
        s = lax.axis_index("s")
        wid = c * 16 + s
        r0 = s * RPT

        pltpu.async_copy(src_hbm.at[wid * nph], src_v.at[0], sem_i)
        pltpu.async_copy(dst_hbm.at[wid * nph], dst_v.at[0], sem_i)

        @pl.when(c == 0)
        def _():
            pltpu.sync_copy(xwp_hbm.at[pl.ds(r0, RPT), :],
                            acc_sh.at[pl.ds(r0, RPT), :])

        @pl.when(c != 0)
        def _():
            pltpu.sync_copy(zeros_hbm, acc_sh.at[pl.ds(r0, RPT), :])

        plsc.subcore_barrier()

        for p in range(nph):
            pb = p % 2
            pltpu.make_async_copy(src_hbm.at[wid * nph + p],
                                  src_v.at[pb], sem_i).wait()
            pltpu.make_async_copy(dst_hbm.at[wid * nph + p],
                                  dst_v.at[pb], sem_i).wait()
            if p + 1 < nph:
                pltpu.async_copy(src_hbm.at[wid * nph + p + 1],
                                 src_v.at[1 - pb], sem_i)
                pltpu.async_copy(dst_hbm.at[wid * nph + p + 1],
                                 dst_v.at[1 - pb], sem_i)

            for g in range(G):
                pltpu.async_copy(xwp_hbm.at[src_v.at[pb, g]],
                                 rows_v.at[g], sg[g])

            def body(jr, carry):
                jbase = jr * ring
                for u in range(ring):
                    j = jbase + u
                    nu = (u + G) % ring
                    pltpu.make_async_copy(xwp_hbm.at[src_v.at[pb, j]],
                                          rows_v.at[u], sg[u]).wait()
                    pltpu.async_copy(rows_v.at[u],
                                     acc_sh.at[dst_v.at[pb, j]],
                                     ss[u], add=True)
                    if u < ring - G:
                        @pl.when(jr > 0)
                        def _(j=j, u=u, nu=nu):
                            pltpu.make_async_copy(
                                rows_v.at[nu],
                                acc_sh.at[dst_v.at[pb, j + G - ring]],
                                ss[nu]).wait()
                    else:
                        pltpu.make_async_copy(
                            rows_v.at[nu],
                            acc_sh.at[dst_v.at[pb, j + G - ring]],
                            ss[nu]).wait()

                    @pl.when(j + G < pc)
                    def _(j=j, u=u, nu=nu):
                        pltpu.async_copy(xwp_hbm.at[src_v.at[pb, j + G]],
                                         rows_v.at[nu], sg[nu])
                return carry

            lax.fori_loop(0, pc // ring, body, 0)
            for j in range(pc - (ring - G), pc):
                pltpu.make_async_copy(rows_v.at[j % ring],
                                      acc_sh.at[dst_v.at[pb, j]],
                                      ss[j % ring]).wait()

        plsc.subcore_barrier()
        pltpu.sync_copy(acc_sh.at[pl.ds(r0, RPT), :],
                        out_hbm.at[c, pl.ds(r0, RPT), :])

    return k(xwp, src3, dst3, zeros_blk)


def _tc_mm1(x, W1, degt):
    def body(x_ref, w_ref, deg_ref, xw_ref, dinv_ref):
        deg = jnp.sum(deg_ref[...], axis=1, keepdims=True) + 1.0
        dinv = lax.rsqrt(deg)
        dinv_ref[...] = jnp.broadcast_to(dinv, (ROWBLK, 16))
        xw = jnp.dot(x_ref[...], w_ref[...],
                     preferred_element_type=jnp.float32)
        xw_ref[...] = xw * dinv

    return pl.pallas_call(
        body,
        grid=(GRID,),
        in_specs=[
            pl.BlockSpec((ROWBLK, 128), lambda i: (i, 0)),
            pl.BlockSpec((128, 128), lambda i: (0, 0)),
            pl.BlockSpec((ROWBLK, NTILES), lambda i: (i, 0)),
        ],
        out_specs=[
            pl.BlockSpec((ROWBLK, 128), lambda i: (i, 0)),
            pl.BlockSpec((ROWBLK, 16), lambda i: (i, 0)),
        ],
        out_shape=[
            jax.ShapeDtypeStruct((NP, 128), jnp.float32),
            jax.ShapeDtypeStruct((NP, 16), jnp.float32),
        ],
    )(x, W1, degt)


def _tc_mm2(acc1, dinvw, b1, W2):
    def body(a_ref, dinv_ref, b_ref, w_ref, out_ref):
        dinv = dinv_ref[...][:, 0:1]
        pre = (a_ref[0] + a_ref[1]) * dinv + b_ref[...]
        h = jnp.maximum(pre, 0.0)
        xw = jnp.dot(h, w_ref[...], preferred_element_type=jnp.float32)
        out_ref[...] = xw * dinv

    return pl.pallas_call(
        body,
        grid=(GRID,),
        in_specs=[
            pl.BlockSpec((2, ROWBLK, 128), lambda i: (0, i, 0)),
            pl.BlockSpec((ROWBLK, 16), lambda i: (i, 0)),
            pl.BlockSpec((1, 128), lambda i: (0, 0)),
            pl.BlockSpec((128, 64), lambda i: (0, 0)),
        ],
        out_specs=pl.BlockSpec((ROWBLK, 64), lambda i: (i, 0)),
        out_shape=jax.ShapeDtypeStruct((NP, 64), jnp.float32),
    )(acc1, dinvw, b1, W2)


def _tc_final(acc2, dinvw, b2, clusterT):
    def body(a_ref, dinv_ref, b_ref, ct_ref, feat_ref, q_ref):
        dinv = dinv_ref[...][:, 0:1]
        f = (a_ref[0] + a_ref[1]) * dinv + b_ref[...]
        feat_ref[...] = f
        ct = ct_ref[...]
        fsq = jnp.sum(f * f, axis=1, keepdims=True)
        csq = jnp.sum(ct * ct, axis=0, keepdims=True)
        fc = jnp.dot(f, ct, preferred_element_type=jnp.float32)
        dist = fsq - 2.0 * fc + csq
        u = 1.0 + dist * 0.5
        r = lax.rsqrt(u)
        qun = r * r * r
        q_ref[...] = qun / jnp.sum(qun, axis=1, keepdims=True)

    return pl.pallas_call(
        body,
        grid=(GRID,),
        in_specs=[
            pl.BlockSpec((2, ROWBLK, 64), lambda i: (0, i, 0)),
            pl.BlockSpec((ROWBLK, 16), lambda i: (i, 0)),
            pl.BlockSpec((1, 64), lambda i: (0, 0)),
            pl.BlockSpec((64, 16), lambda i: (0, 0)),
        ],
        out_specs=[
            pl.BlockSpec((ROWBLK, 64), lambda i: (i, 0)),
            pl.BlockSpec((ROWBLK, 16), lambda i: (i, 0)),
        ],
        out_shape=[
            jax.ShapeDtypeStruct((NP, 64), jnp.float32),
            jax.ShapeDtypeStruct((NP, 16), jnp.float32),
        ],
    )(acc2, dinvw, b2, clusterT)


def kernel(x, edge_index, W1, b1, W2, b2, cluster):
    src = edge_index[0].astype(jnp.int32)
    dst = edge_index[1].astype(jnp.int32)
    x_p = jnp.zeros((NP, 128), jnp.float32).at[:N].set(x)

    zeros128 = jnp.zeros((RPT, 128), jnp.float32)
    zeros64 = jnp.zeros((RPT, 64), jnp.float32)

    src3a = src.reshape(NTILES * NPH1, PC1, B1)
    dst3a = dst.reshape(NTILES * NPH1, PC1, B1)
    src3b = src.reshape(NTILES * NPH2, PC2, B2)
    dst3b = dst.reshape(NTILES * NPH2, PC2, B2)
    degt = _sc_degree(dst).T
    xw1p, dinvw = _tc_mm1(x_p, W1, degt)
    acc1 = _sc_pass(128, B1, PC1, NPH1, R1, xw1p, src3a, dst3a, zeros128)
    xw2p = _tc_mm2(acc1, dinvw, b1.reshape(1, 128), W2)
    acc2 = _sc_pass(64, B2, PC2, NPH2, R2, xw2p, src3b, dst3b, zeros64)
    feature, q = _tc_final(acc2, dinvw, b2.reshape(1, 64), cluster.T)
    return (feature[:N], q[:N])


# speedup vs baseline: 31.2784x; 1.0167x over previous
import functools

import jax
import jax.numpy as jnp
from jax import lax
from jax.experimental import pallas as pl
from jax.experimental.pallas import tpu as pltpu
from jax.experimental.pallas import tpu_sc as plsc

N = 10000
NP = 10240
E = 320000
NTILES = 32
EPT = E // NTILES
B1, PC1, NPH1, R1 = 50, 40, 5, 4
B2, PC2, NPH2, R2 = 200, 10, 5, 2
RPT = NP // 16
ROWBLK = 1024
GRID = NP // ROWBLK


def _sc_mesh():
    return plsc.VectorSubcoreMesh(core_axis_name="c", subcore_axis_name="s",
                                  num_cores=2, num_subcores=16)


def _sc_degree(dst):
    @functools.partial(
        pl.kernel,
        out_type=jax.ShapeDtypeStruct((NTILES, NP), jnp.float32),
        mesh=_sc_mesh(),
        compiler_params=pltpu.CompilerParams(needs_layout_passes=False),
        scratch_types=[
            pltpu.VMEM((NP,), jnp.float32),
            pltpu.VMEM((EPT,), jnp.int32),
        ],
    )
    def k(dst_hbm, out_hbm, deg_v, idx_d):
        c = lax.axis_index("c")
        s = lax.axis_index("s")
        wid = c * 16 + s
        ebase = pl.multiple_of(wid * EPT, 8)
        pltpu.sync_copy(dst_hbm.at[pl.ds(ebase, EPT)], idx_d)
        zeros = jnp.zeros((16,), jnp.float32)

        def zbody(j, carry):
            deg_v[pl.ds(j * 16, 16)] = zeros
            return carry

        lax.fori_loop(0, NP // 16, zbody, 0)
        ones = jnp.ones((16,), jnp.float32)

        def body(j, carry):
            idx16 = idx_d[pl.ds(j * 16, 16)]
            plsc.addupdate_scatter(deg_v, [idx16], ones)
            return carry

        lax.fori_loop(0, EPT // 16, body, 0, unroll=8)
        pltpu.sync_copy(deg_v, out_hbm.at[wid])

    return k(dst)


def _sc_pass(d, bb, pc, nph, ring, xwp, src3, dst3, zeros_blk):
    G = ring - 2 if ring > 2 else 1

    @functools.partial(
        pl.kernel,
        out_type=jax.ShapeDtypeStruct((2, NP, d), jnp.float32),
        mesh=_sc_mesh(),
        compiler_params=pltpu.CompilerParams(
            use_tc_tiling_on_sc=(d % 128 == 0)),
        scratch_types=[
            pltpu.VMEM_SHARED((NP, d), jnp.float32),
            pltpu.VMEM((2, pc, bb), jnp.int32),
            pltpu.VMEM((2, pc, bb), jnp.int32),
            pltpu.VMEM((ring, bb, d), jnp.float32),
            [pltpu.SemaphoreType.DMA] * ring,
            [pltpu.SemaphoreType.DMA] * ring,
            pltpu.SemaphoreType.DMA,
        ],
    )
    def k(xwp_hbm, src_hbm, dst_hbm, zeros_hbm, out_hbm,
          acc_sh, src_v, dst_v, rows_v, sg, ss, sem_i):
        c = lax.axis_index("c")
        s = lax.axis_index("s")
        wid = c * 16 + s
        r0 = s * RPT

        pltpu.async_copy(src_hbm.at[wid * nph], src_v.at[0], sem_i)
        pltpu.async_copy(dst_hbm.at[wid * nph], dst_v.at[0], sem_i)

        @pl.when(c == 0)
        def _():
            pltpu.sync_copy(xwp_hbm.at[pl.ds(r0, RPT), :],
                            acc_sh.at[pl.ds(r0, RPT), :])

        @pl.when(c != 0)
        def _():
            pltpu.sync_copy(zeros_hbm, acc_sh.at[pl.ds(r0, RPT), :])

        plsc.subcore_barrier()

        for p in range(nph):
            pb = p % 2
            pltpu.make_async_copy(src_hbm.at[wid * nph + p],
                                  src_v.at[pb], sem_i).wait()
            pltpu.make_async_copy(dst_hbm.at[wid * nph + p],
                                  dst_v.at[pb], sem_i).wait()
            if p + 1 < nph:
                pltpu.async_copy(src_hbm.at[wid * nph + p + 1],
                                 src_v.at[1 - pb], sem_i)
                pltpu.async_copy(dst_hbm.at[wid * nph + p + 1],
                                 dst_v.at[1 - pb], sem_i)

            for g in range(G):
                pltpu.async_copy(xwp_hbm.at[src_v.at[pb, g]],
                                 rows_v.at[g], sg[g])

            def body(jr, carry):
                jbase = jr * ring
                for u in range(ring):
                    j = jbase + u
                    nu = (u + G) % ring
                    pltpu.make_async_copy(xwp_hbm.at[src_v.at[pb, j]],
                                          rows_v.at[u], sg[u]).wait()
                    pltpu.async_copy(rows_v.at[u],
                                     acc_sh.at[dst_v.at[pb, j]],
                                     ss[u], add=True)
                    if u < ring - G:
                        @pl.when(jr > 0)
                        def _(j=j, u=u, nu=nu):
                            pltpu.make_async_copy(
                                rows_v.at[nu],
                                acc_sh.at[dst_v.at[pb, j + G - ring]],
                                ss[nu]).wait()
                    else:
                        pltpu.make_async_copy(
                            rows_v.at[nu],
                            acc_sh.at[dst_v.at[pb, j + G - ring]],
                            ss[nu]).wait()

                    @pl.when(j + G < pc)
                    def _(j=j, u=u, nu=nu):
                        pltpu.async_copy(xwp_hbm.at[src_v.at[pb, j + G]],
                                         rows_v.at[nu], sg[nu])
                return carry

            lax.fori_loop(0, pc // ring, body, 0)
            for j in range(pc - (ring - G), pc):
                pltpu.make_async_copy(rows_v.at[j % ring],
                                      acc_sh.at[dst_v.at[pb, j]],
                                      ss[j % ring]).wait()

        plsc.subcore_barrier()
        pltpu.sync_copy(acc_sh.at[pl.ds(r0, RPT), :],
                        out_hbm.at[c, pl.ds(r0, RPT), :])

    return k(xwp, src3, dst3, zeros_blk)


def _tc_mm1(x, W1, degp):
    def body(x_ref, w_ref, deg_ref, xw_ref, dinv_ref):
        degt = jnp.transpose(deg_ref[...])
        deg = jnp.sum(degt, axis=1, keepdims=True) + 1.0
        dinv = lax.rsqrt(deg)
        dinv_ref[...] = jnp.broadcast_to(dinv, (ROWBLK, 16))
        xw = jnp.dot(x_ref[...], w_ref[...],
                     preferred_element_type=jnp.float32)
        xw_ref[...] = xw * dinv

    return pl.pallas_call(
        body,
        grid=(GRID,),
        in_specs=[
            pl.BlockSpec((ROWBLK, 128), lambda i: (i, 0)),
            pl.BlockSpec((128, 128), lambda i: (0, 0)),
            pl.BlockSpec((NTILES, ROWBLK), lambda i: (0, i)),
        ],
        out_specs=[
            pl.BlockSpec((ROWBLK, 128), lambda i: (i, 0)),
            pl.BlockSpec((ROWBLK, 16), lambda i: (i, 0)),
        ],
        out_shape=[
            jax.ShapeDtypeStruct((NP, 128), jnp.float32),
            jax.ShapeDtypeStruct((NP, 16), jnp.float32),
        ],
    )(x, W1, degp)


def _tc_mm2(acc1, dinvw, b1, W2):
    def body(a_ref, dinv_ref, b_ref, w_ref, out_ref):
        dinv = dinv_ref[...][:, 0:1]
        pre = (a_ref[0] + a_ref[1]) * dinv + b_ref[...]
        h = jnp.maximum(pre, 0.0)
        xw = jnp.dot(h, w_ref[...], preferred_element_type=jnp.float32)
        out_ref[...] = xw * dinv

    return pl.pallas_call(
        body,
        grid=(GRID,),
        in_specs=[
            pl.BlockSpec((2, ROWBLK, 128), lambda i: (0, i, 0)),
            pl.BlockSpec((ROWBLK, 16), lambda i: (i, 0)),
            pl.BlockSpec((1, 128), lambda i: (0, 0)),
            pl.BlockSpec((128, 64), lambda i: (0, 0)),
        ],
        out_specs=pl.BlockSpec((ROWBLK, 64), lambda i: (i, 0)),
        out_shape=jax.ShapeDtypeStruct((NP, 64), jnp.float32),
    )(acc1, dinvw, b1, W2)


def _tc_final(acc2, dinvw, b2, clusterT):
    def body(a_ref, dinv_ref, b_ref, ct_ref, feat_ref, q_ref):
        dinv = dinv_ref[...][:, 0:1]
        f = (a_ref[0] + a_ref[1]) * dinv + b_ref[...]
        feat_ref[...] = f
        ct = ct_ref[...]
        fsq = jnp.sum(f * f, axis=1, keepdims=True)
        csq = jnp.sum(ct * ct, axis=0, keepdims=True)
        fc = jnp.dot(f, ct, preferred_element_type=jnp.float32)
        dist = fsq - 2.0 * fc + csq
        u = 1.0 + dist * 0.5
        r = lax.rsqrt(u)
        qun = r * r * r
        q_ref[...] = qun / jnp.sum(qun, axis=1, keepdims=True)

    return pl.pallas_call(
        body,
        grid=(GRID,),
        in_specs=[
            pl.BlockSpec((2, ROWBLK, 64), lambda i: (0, i, 0)),
            pl.BlockSpec((ROWBLK, 16), lambda i: (i, 0)),
            pl.BlockSpec((1, 64), lambda i: (0, 0)),
            pl.BlockSpec((64, 16), lambda i: (0, 0)),
        ],
        out_specs=[
            pl.BlockSpec((ROWBLK, 64), lambda i: (i, 0)),
            pl.BlockSpec((ROWBLK, 16), lambda i: (i, 0)),
        ],
        out_shape=[
            jax.ShapeDtypeStruct((NP, 64), jnp.float32),
            jax.ShapeDtypeStruct((NP, 16), jnp.float32),
        ],
    )(acc2, dinvw, b2, clusterT)


def kernel(x, edge_index, W1, b1, W2, b2, cluster):
    src = edge_index[0].astype(jnp.int32)
    dst = edge_index[1].astype(jnp.int32)
    x_p = jnp.zeros((NP, 128), jnp.float32).at[:N].set(x)

    zeros128 = jnp.zeros((RPT, 128), jnp.float32)
    zeros64 = jnp.zeros((RPT, 64), jnp.float32)

    src3a = src.reshape(NTILES * NPH1, PC1, B1)
    dst3a = dst.reshape(NTILES * NPH1, PC1, B1)
    src3b = src.reshape(NTILES * NPH2, PC2, B2)
    dst3b = dst.reshape(NTILES * NPH2, PC2, B2)
    degp = _sc_degree(dst)
    xw1p, dinvw = _tc_mm1(x_p, W1, degp)
    acc1 = _sc_pass(128, B1, PC1, NPH1, R1, xw1p, src3a, dst3a, zeros128)
    xw2p = _tc_mm2(acc1, dinvw, b1.reshape(1, 128), W2)
    acc2 = _sc_pass(64, B2, PC2, NPH2, R2, xw2p, src3b, dst3b, zeros64)
    feature, q = _tc_final(acc2, dinvw, b2.reshape(1, 64), cluster.T)
    return (feature[:N], q[:N])


# speedup vs baseline: 35.0567x; 1.1208x over previous
import functools

import jax
import jax.numpy as jnp
from jax import lax
from jax.experimental import pallas as pl
from jax.experimental.pallas import tpu as pltpu
from jax.experimental.pallas import tpu_sc as plsc

N = 10000
NP = 10240
E = 320000
NTILES = 32
EPT = E // NTILES
B1, PC1, NPH1, R1 = 50, 40, 5, 4
B2, PC2, NPH2, R2 = 100, 20, 5, 4
RPT = NP // 16
ROWBLK = 1024
GRID = NP // ROWBLK


def _sc_mesh():
    return plsc.VectorSubcoreMesh(core_axis_name="c", subcore_axis_name="s",
                                  num_cores=2, num_subcores=16)


def _sc_degree(dst):
    @functools.partial(
        pl.kernel,
        out_type=jax.ShapeDtypeStruct((NTILES, NP), jnp.float32),
        mesh=_sc_mesh(),
        compiler_params=pltpu.CompilerParams(needs_layout_passes=False),
        scratch_types=[
            pltpu.VMEM((NP,), jnp.float32),
            pltpu.VMEM((EPT,), jnp.int32),
        ],
    )
    def k(dst_hbm, out_hbm, deg_v, idx_d):
        c = lax.axis_index("c")
        s = lax.axis_index("s")
        wid = c * 16 + s
        ebase = pl.multiple_of(wid * EPT, 8)
        pltpu.sync_copy(dst_hbm.at[pl.ds(ebase, EPT)], idx_d)
        zeros = jnp.zeros((16,), jnp.float32)

        def zbody(j, carry):
            deg_v[pl.ds(j * 16, 16)] = zeros
            return carry

        lax.fori_loop(0, NP // 16, zbody, 0)
        ones = jnp.ones((16,), jnp.float32)

        def body(j, carry):
            idx16 = idx_d[pl.ds(j * 16, 16)]
            plsc.addupdate_scatter(deg_v, [idx16], ones)
            return carry

        lax.fori_loop(0, EPT // 16, body, 0, unroll=8)
        pltpu.sync_copy(deg_v, out_hbm.at[wid])

    return k(dst)


def _sc_pass(d, bb, pc, nph, ring, xwp, src3, dst3, zeros_blk):
    G = ring - 1 if ring > 2 else 1

    @functools.partial(
        pl.kernel,
        out_type=jax.ShapeDtypeStruct((2, NP, d), jnp.float32),
        mesh=_sc_mesh(),
        compiler_params=pltpu.CompilerParams(
            use_tc_tiling_on_sc=(d % 128 == 0)),
        scratch_types=[
            pltpu.VMEM_SHARED((NP, d), jnp.float32),
            pltpu.VMEM((2, pc, bb), jnp.int32),
            pltpu.VMEM((2, pc, bb), jnp.int32),
            pltpu.VMEM((ring, bb, d), jnp.float32),
            [pltpu.SemaphoreType.DMA] * ring,
            [pltpu.SemaphoreType.DMA] * ring,
            pltpu.SemaphoreType.DMA,
        ],
    )
    def k(xwp_hbm, src_hbm, dst_hbm, zeros_hbm, out_hbm,
          acc_sh, src_v, dst_v, rows_v, sg, ss, sem_i):
        c = lax.axis_index("c")
        s = lax.axis_index("s")
        wid = c * 16 + s
        r0 = s * RPT

        pltpu.async_copy(src_hbm.at[wid * nph], src_v.at[0], sem_i)
        pltpu.async_copy(dst_hbm.at[wid * nph], dst_v.at[0], sem_i)

        @pl.when(c == 0)
        def _():
            pltpu.sync_copy(xwp_hbm.at[pl.ds(r0, RPT), :],
                            acc_sh.at[pl.ds(r0, RPT), :])

        @pl.when(c != 0)
        def _():
            pltpu.sync_copy(zeros_hbm, acc_sh.at[pl.ds(r0, RPT), :])

        plsc.subcore_barrier()

        for p in range(nph):
            pb = p % 2
            pltpu.make_async_copy(src_hbm.at[wid * nph + p],
                                  src_v.at[pb], sem_i).wait()
            pltpu.make_async_copy(dst_hbm.at[wid * nph + p],
                                  dst_v.at[pb], sem_i).wait()
            if p + 1 < nph:
                pltpu.async_copy(src_hbm.at[wid * nph + p + 1],
                                 src_v.at[1 - pb], sem_i)
                pltpu.async_copy(dst_hbm.at[wid * nph + p + 1],
                                 dst_v.at[1 - pb], sem_i)

            for g in range(G):
                pltpu.async_copy(xwp_hbm.at[src_v.at[pb, g]],
                                 rows_v.at[g], sg[g])

            def body(jr, carry):
                jbase = jr * ring
                for u in range(ring):
                    j = jbase + u
                    nu = (u + G) % ring
                    pltpu.make_async_copy(xwp_hbm.at[src_v.at[pb, j]],
                                          rows_v.at[u], sg[u]).wait()
                    pltpu.async_copy(rows_v.at[u],
                                     acc_sh.at[dst_v.at[pb, j]],
                                     ss[u], add=True)
                    if u < ring - G:
                        @pl.when(jr > 0)
                        def _(j=j, u=u, nu=nu):
                            pltpu.make_async_copy(
                                rows_v.at[nu],
                                acc_sh.at[dst_v.at[pb, j + G - ring]],
                                ss[nu]).wait()
                    else:
                        pltpu.make_async_copy(
                            rows_v.at[nu],
                            acc_sh.at[dst_v.at[pb, j + G - ring]],
                            ss[nu]).wait()

                    @pl.when(j + G < pc)
                    def _(j=j, u=u, nu=nu):
                        pltpu.async_copy(xwp_hbm.at[src_v.at[pb, j + G]],
                                         rows_v.at[nu], sg[nu])
                return carry

            lax.fori_loop(0, pc // ring, body, 0)
            for j in range(pc - (ring - G), pc):
                pltpu.make_async_copy(rows_v.at[j % ring],
                                      acc_sh.at[dst_v.at[pb, j]],
                                      ss[j % ring]).wait()

        plsc.subcore_barrier()
        pltpu.sync_copy(acc_sh.at[pl.ds(r0, RPT), :],
                        out_hbm.at[c, pl.ds(r0, RPT), :])

    return k(xwp, src3, dst3, zeros_blk)


def _tc_mm1(x, W1, degp):
    def body(x_ref, w_ref, deg_ref, xw_ref, dinv_ref):
        degt = jnp.transpose(deg_ref[...])
        deg = jnp.sum(degt, axis=1, keepdims=True) + 1.0
        dinv = lax.rsqrt(deg)
        dinv_ref[...] = jnp.broadcast_to(dinv, (ROWBLK, 16))
        xw = jnp.dot(x_ref[...], w_ref[...],
                     preferred_element_type=jnp.float32)
        xw_ref[...] = xw * dinv

    return pl.pallas_call(
        body,
        grid=(GRID,),
        in_specs=[
            pl.BlockSpec((ROWBLK, 128), lambda i: (i, 0)),
            pl.BlockSpec((128, 128), lambda i: (0, 0)),
            pl.BlockSpec((NTILES, ROWBLK), lambda i: (0, i)),
        ],
        out_specs=[
            pl.BlockSpec((ROWBLK, 128), lambda i: (i, 0)),
            pl.BlockSpec((ROWBLK, 16), lambda i: (i, 0)),
        ],
        out_shape=[
            jax.ShapeDtypeStruct((NP, 128), jnp.float32),
            jax.ShapeDtypeStruct((NP, 16), jnp.float32),
        ],
    )(x, W1, degp)


def _tc_mm2(acc1, dinvw, b1, W2):
    def body(a_ref, dinv_ref, b_ref, w_ref, out_ref):
        dinv = dinv_ref[...][:, 0:1]
        pre = (a_ref[0] + a_ref[1]) * dinv + b_ref[...]
        h = jnp.maximum(pre, 0.0)
        xw = jnp.dot(h, w_ref[...], preferred_element_type=jnp.float32)
        out_ref[...] = xw * dinv

    return pl.pallas_call(
        body,
        grid=(GRID,),
        in_specs=[
            pl.BlockSpec((2, ROWBLK, 128), lambda i: (0, i, 0)),
            pl.BlockSpec((ROWBLK, 16), lambda i: (i, 0)),
            pl.BlockSpec((1, 128), lambda i: (0, 0)),
            pl.BlockSpec((128, 64), lambda i: (0, 0)),
        ],
        out_specs=pl.BlockSpec((ROWBLK, 64), lambda i: (i, 0)),
        out_shape=jax.ShapeDtypeStruct((NP, 64), jnp.float32),
    )(acc1, dinvw, b1, W2)


def _tc_final(acc2, dinvw, b2, clusterT):
    def body(a_ref, dinv_ref, b_ref, ct_ref, feat_ref, q_ref):
        dinv = dinv_ref[...][:, 0:1]
        f = (a_ref[0] + a_ref[1]) * dinv + b_ref[...]
        feat_ref[...] = f
        ct = ct_ref[...]
        fsq = jnp.sum(f * f, axis=1, keepdims=True)
        csq = jnp.sum(ct * ct, axis=0, keepdims=True)
        fc = jnp.dot(f, ct, preferred_element_type=jnp.float32)
        dist = fsq - 2.0 * fc + csq
        u = 1.0 + dist * 0.5
        r = lax.rsqrt(u)
        qun = r * r * r
        q_ref[...] = qun / jnp.sum(qun, axis=1, keepdims=True)

    return pl.pallas_call(
        body,
        grid=(GRID,),
        in_specs=[
            pl.BlockSpec((2, ROWBLK, 64), lambda i: (0, i, 0)),
            pl.BlockSpec((ROWBLK, 16), lambda i: (i, 0)),
            pl.BlockSpec((1, 64), lambda i: (0, 0)),
            pl.BlockSpec((64, 16), lambda i: (0, 0)),
        ],
        out_specs=[
            pl.BlockSpec((ROWBLK, 64), lambda i: (i, 0)),
            pl.BlockSpec((ROWBLK, 16), lambda i: (i, 0)),
        ],
        out_shape=[
            jax.ShapeDtypeStruct((NP, 64), jnp.float32),
            jax.ShapeDtypeStruct((NP, 16), jnp.float32),
        ],
    )(acc2, dinvw, b2, clusterT)


def kernel(x, edge_index, W1, b1, W2, b2, cluster):
    src = edge_index[0].astype(jnp.int32)
    dst = edge_index[1].astype(jnp.int32)
    x_p = jnp.zeros((NP, 128), jnp.float32).at[:N].set(x)

    zeros128 = jnp.zeros((RPT, 128), jnp.float32)
    zeros64 = jnp.zeros((RPT, 64), jnp.float32)

    src3a = src.reshape(NTILES * NPH1, PC1, B1)
    dst3a = dst.reshape(NTILES * NPH1, PC1, B1)
    src3b = src.reshape(NTILES * NPH2, PC2, B2)
    dst3b = dst.reshape(NTILES * NPH2, PC2, B2)
    degp = _sc_degree(dst)
    xw1p, dinvw = _tc_mm1(x_p, W1, degp)
    acc1 = _sc_pass(128, B1, PC1, NPH1, R1, xw1p, src3a, dst3a, zeros128)
    xw2p = _tc_mm2(acc1, dinvw, b1.reshape(1, 128), W2)
    acc2 = _sc_pass(64, B2, PC2, NPH2, R2, xw2p, src3b, dst3b, zeros64)
    feature, q = _tc_final(acc2, dinvw, b2.reshape(1, 64), cluster.T)
    return (feature[:N], q[:N])


# speedup vs baseline: 35.5129x; 1.0130x over previous
import functools

import jax
import jax.numpy as jnp
from jax import lax
from jax.experimental import pallas as pl
from jax.experimental.pallas import tpu as pltpu
from jax.experimental.pallas import tpu_sc as plsc

N = 10000
NP = 10240
E = 320000
NTILES = 32
EPT = E // NTILES
B1, PC1, NPH1, R1 = 50, 40, 5, 4
B2, PC2, NPH2, R2 = 100, 20, 5, 4
RPT = NP // 16
ROWBLK = 1024
GRID = NP // ROWBLK
NRB = 1000


def _sc_mesh():
    return plsc.VectorSubcoreMesh(core_axis_name="c", subcore_axis_name="s",
                                  num_cores=2, num_subcores=16)


def _sc_degree(dst):
    @functools.partial(
        pl.kernel,
        out_type=jax.ShapeDtypeStruct((NTILES, NP), jnp.float32),
        mesh=_sc_mesh(),
        compiler_params=pltpu.CompilerParams(needs_layout_passes=False),
        scratch_types=[
            pltpu.VMEM((NP,), jnp.float32),
            pltpu.VMEM((EPT,), jnp.int32),
        ],
    )
    def k(dst_hbm, out_hbm, deg_v, idx_d):
        c = lax.axis_index("c")
        s = lax.axis_index("s")
        wid = c * 16 + s
        ebase = pl.multiple_of(wid * EPT, 8)
        pltpu.sync_copy(dst_hbm.at[pl.ds(ebase, EPT)], idx_d)
        zeros = jnp.zeros((16,), jnp.float32)

        def zbody(j, carry):
            deg_v[pl.ds(j * 16, 16)] = zeros
            return carry

        lax.fori_loop(0, NP // 16, zbody, 0)
        ones = jnp.ones((16,), jnp.float32)

        def body(j, carry):
            idx16 = idx_d[pl.ds(j * 16, 16)]
            plsc.addupdate_scatter(deg_v, [idx16], ones)
            return carry

        lax.fori_loop(0, EPT // 16, body, 0, unroll=8)
        pltpu.sync_copy(deg_v, out_hbm.at[wid])

    return k(dst)


def _sc_pass(d, bb, pc, nph, ring, xwp, src3, dst3, zeros_blk):
    G = ring - 1 if ring > 2 else 1

    @functools.partial(
        pl.kernel,
        out_type=jax.ShapeDtypeStruct((2, NP, d), jnp.float32),
        mesh=_sc_mesh(),
        compiler_params=pltpu.CompilerParams(
            use_tc_tiling_on_sc=(d % 128 == 0)),
        scratch_types=[
            pltpu.VMEM_SHARED((NP, d), jnp.float32),
            pltpu.VMEM((2, pc, bb), jnp.int32),
            pltpu.VMEM((2, pc, bb), jnp.int32),
            pltpu.VMEM((ring, bb, d), jnp.float32),
            [pltpu.SemaphoreType.DMA] * ring,
            [pltpu.SemaphoreType.DMA] * ring,
            pltpu.SemaphoreType.DMA,
        ],
    )
    def k(xwp_hbm, src_hbm, dst_hbm, zeros_hbm, out_hbm,
          acc_sh, src_v, dst_v, rows_v, sg, ss, sem_i):
        c = lax.axis_index("c")
        s = lax.axis_index("s")
        wid = c * 16 + s
        r0 = s * RPT
        nrows = xwp_hbm.shape[0]

        pltpu.async_copy(src_hbm.at[wid * nph], src_v.at[0], sem_i)
        pltpu.async_copy(dst_hbm.at[wid * nph], dst_v.at[0], sem_i)

        if nrows == NP:
            @pl.when(c == 0)
            def _():
                pltpu.sync_copy(xwp_hbm.at[pl.ds(r0, RPT), :],
                                acc_sh.at[pl.ds(r0, RPT), :])
        else:
            tail = N - 15 * RPT
            @pl.when(jnp.logical_and(c == 0, s < 15))
            def _():
                pltpu.sync_copy(xwp_hbm.at[pl.ds(r0, RPT), :],
                                acc_sh.at[pl.ds(r0, RPT), :])

            @pl.when(jnp.logical_and(c == 0, s == 15))
            def _():
                pltpu.sync_copy(xwp_hbm.at[pl.ds(15 * RPT, tail), :],
                                acc_sh.at[pl.ds(15 * RPT, tail), :])
                pltpu.sync_copy(zeros_hbm.at[pl.ds(0, NP - N), :],
                                acc_sh.at[pl.ds(N, NP - N), :])

        @pl.when(c != 0)
        def _():
            pltpu.sync_copy(zeros_hbm, acc_sh.at[pl.ds(r0, RPT), :])

        plsc.subcore_barrier()

        for p in range(nph):
            pb = p % 2
            pltpu.make_async_copy(src_hbm.at[wid * nph + p],
                                  src_v.at[pb], sem_i).wait()
            pltpu.make_async_copy(dst_hbm.at[wid * nph + p],
                                  dst_v.at[pb], sem_i).wait()
            if p + 1 < nph:
                pltpu.async_copy(src_hbm.at[wid * nph + p + 1],
                                 src_v.at[1 - pb], sem_i)
                pltpu.async_copy(dst_hbm.at[wid * nph + p + 1],
                                 dst_v.at[1 - pb], sem_i)

            for g in range(G):
                pltpu.async_copy(xwp_hbm.at[src_v.at[pb, g]],
                                 rows_v.at[g], sg[g])

            def body(jr, carry):
                jbase = jr * ring
                for u in range(ring):
                    j = jbase + u
                    nu = (u + G) % ring
                    pltpu.make_async_copy(xwp_hbm.at[src_v.at[pb, j]],
                                          rows_v.at[u], sg[u]).wait()
                    pltpu.async_copy(rows_v.at[u],
                                     acc_sh.at[dst_v.at[pb, j]],
                                     ss[u], add=True)
                    if u < ring - G:
                        @pl.when(jr > 0)
                        def _(j=j, u=u, nu=nu):
                            pltpu.make_async_copy(
                                rows_v.at[nu],
                                acc_sh.at[dst_v.at[pb, j + G - ring]],
                                ss[nu]).wait()
                    else:
                        pltpu.make_async_copy(
                            rows_v.at[nu],
                            acc_sh.at[dst_v.at[pb, j + G - ring]],
                            ss[nu]).wait()

                    @pl.when(j + G < pc)
                    def _(j=j, u=u, nu=nu):
                        pltpu.async_copy(xwp_hbm.at[src_v.at[pb, j + G]],
                                         rows_v.at[nu], sg[nu])
                return carry

            lax.fori_loop(0, pc // ring, body, 0)
            for j in range(pc - (ring - G), pc):
                pltpu.make_async_copy(rows_v.at[j % ring],
                                      acc_sh.at[dst_v.at[pb, j]],
                                      ss[j % ring]).wait()

        plsc.subcore_barrier()
        pltpu.sync_copy(acc_sh.at[pl.ds(r0, RPT), :],
                        out_hbm.at[c, pl.ds(r0, RPT), :])

    return k(xwp, src3, dst3, zeros_blk)


def _tc_mm1(x, W1, degp):
    def body(x_ref, w_ref, deg_ref, xw_ref, dinv_ref):
        degt = jnp.transpose(deg_ref[...])
        deg = jnp.sum(degt, axis=1, keepdims=True) + 1.0
        dinv = lax.rsqrt(deg)
        dinv_ref[...] = jnp.broadcast_to(dinv, (ROWBLK, 16))
        xw = jnp.dot(x_ref[...], w_ref[...],
                     preferred_element_type=jnp.float32)
        xw_ref[...] = xw * dinv

    return pl.pallas_call(
        body,
        grid=(GRID,),
        in_specs=[
            pl.BlockSpec((ROWBLK, 128), lambda i: (i, 0)),
            pl.BlockSpec((128, 128), lambda i: (0, 0)),
            pl.BlockSpec((NTILES, ROWBLK), lambda i: (0, i)),
        ],
        out_specs=[
            pl.BlockSpec((ROWBLK, 128), lambda i: (i, 0)),
            pl.BlockSpec((ROWBLK, 16), lambda i: (i, 0)),
        ],
        out_shape=[
            jax.ShapeDtypeStruct((NP, 128), jnp.float32),
            jax.ShapeDtypeStruct((NP, 16), jnp.float32),
        ],
    )(x, W1, degp)


def _tc_mm2(acc1, dinvw, b1, W2):
    def body(a_ref, dinv_ref, b_ref, w_ref, out_ref):
        dinv = dinv_ref[...][:, 0:1]
        pre = (a_ref[0] + a_ref[1]) * dinv + b_ref[...]
        h = jnp.maximum(pre, 0.0)
        xw = jnp.dot(h, w_ref[...], preferred_element_type=jnp.float32)
        out_ref[...] = xw * dinv

    return pl.pallas_call(
        body,
        grid=(N // NRB,),
        in_specs=[
            pl.BlockSpec((2, NRB, 128), lambda i: (0, i, 0)),
            pl.BlockSpec((NRB, 16), lambda i: (i, 0)),
            pl.BlockSpec((1, 128), lambda i: (0, 0)),
            pl.BlockSpec((128, 64), lambda i: (0, 0)),
        ],
        out_specs=pl.BlockSpec((NRB, 64), lambda i: (i, 0)),
        out_shape=jax.ShapeDtypeStruct((N, 64), jnp.float32),
    )(acc1, dinvw, b1, W2)


def _tc_final(acc2, dinvw, b2, clusterT):
    def body(a_ref, dinv_ref, b_ref, ct_ref, feat_ref, q_ref):
        dinv = dinv_ref[...][:, 0:1]
        f = (a_ref[0] + a_ref[1]) * dinv + b_ref[...]
        feat_ref[...] = f
        ct = ct_ref[...]
        fsq = jnp.sum(f * f, axis=1, keepdims=True)
        csq = jnp.sum(ct * ct, axis=0, keepdims=True)
        fc = jnp.dot(f, ct, preferred_element_type=jnp.float32)
        dist = fsq - 2.0 * fc + csq
        u = 1.0 + dist * 0.5
        r = lax.rsqrt(u)
        qun = r * r * r
        q_ref[...] = qun / jnp.sum(qun, axis=1, keepdims=True)

    return pl.pallas_call(
        body,
        grid=(N // NRB,),
        in_specs=[
            pl.BlockSpec((2, NRB, 64), lambda i: (0, i, 0)),
            pl.BlockSpec((NRB, 16), lambda i: (i, 0)),
            pl.BlockSpec((1, 64), lambda i: (0, 0)),
            pl.BlockSpec((64, 16), lambda i: (0, 0)),
        ],
        out_specs=[
            pl.BlockSpec((NRB, 64), lambda i: (i, 0)),
            pl.BlockSpec((NRB, 16), lambda i: (i, 0)),
        ],
        out_shape=[
            jax.ShapeDtypeStruct((N, 64), jnp.float32),
            jax.ShapeDtypeStruct((N, 16), jnp.float32),
        ],
    )(acc2, dinvw, b2, clusterT)


def kernel(x, edge_index, W1, b1, W2, b2, cluster):
    src = edge_index[0].astype(jnp.int32)
    dst = edge_index[1].astype(jnp.int32)
    x_p = jnp.zeros((NP, 128), jnp.float32).at[:N].set(x)

    zeros128 = jnp.zeros((RPT, 128), jnp.float32)
    zeros64 = jnp.zeros((RPT, 64), jnp.float32)

    src3a = src.reshape(NTILES * NPH1, PC1, B1)
    dst3a = dst.reshape(NTILES * NPH1, PC1, B1)
    src3b = src.reshape(NTILES * NPH2, PC2, B2)
    dst3b = dst.reshape(NTILES * NPH2, PC2, B2)
    degp = _sc_degree(dst)
    xw1p, dinvw = _tc_mm1(x_p, W1, degp)
    acc1 = _sc_pass(128, B1, PC1, NPH1, R1, xw1p, src3a, dst3a, zeros128)
    xw2p = _tc_mm2(acc1, dinvw, b1.reshape(1, 128), W2)
    acc2 = _sc_pass(64, B2, PC2, NPH2, R2, xw2p, src3b, dst3b, zeros64)
    feature, q = _tc_final(acc2, dinvw, b2.reshape(1, 64), cluster.T)
    return (feature, q)


# speedup vs baseline: 35.9428x; 1.0121x over previous
import functools

import jax
import jax.numpy as jnp
from jax import lax
from jax.experimental import pallas as pl
from jax.experimental.pallas import tpu as pltpu
from jax.experimental.pallas import tpu_sc as plsc

N = 10000
NP = 10240
E = 320000
NTILES = 32
EPT = E // NTILES
B1, PC1, NPH1, R1 = 50, 40, 5, 4
B2, PC2, NPH2, R2 = 100, 20, 5, 5
RPT = NP // 16
ROWBLK = 1024
GRID = NP // ROWBLK
NRB = 1000


def _sc_mesh():
    return plsc.VectorSubcoreMesh(core_axis_name="c", subcore_axis_name="s",
                                  num_cores=2, num_subcores=16)


def _sc_degree(dst):
    @functools.partial(
        pl.kernel,
        out_type=jax.ShapeDtypeStruct((NTILES, NP), jnp.float32),
        mesh=_sc_mesh(),
        compiler_params=pltpu.CompilerParams(needs_layout_passes=False),
        scratch_types=[
            pltpu.VMEM((NP,), jnp.float32),
            pltpu.VMEM((EPT,), jnp.int32),
        ],
    )
    def k(dst_hbm, out_hbm, deg_v, idx_d):
        c = lax.axis_index("c")
        s = lax.axis_index("s")
        wid = c * 16 + s
        ebase = pl.multiple_of(wid * EPT, 8)
        pltpu.sync_copy(dst_hbm.at[pl.ds(ebase, EPT)], idx_d)
        zeros = jnp.zeros((16,), jnp.float32)

        def zbody(j, carry):
            deg_v[pl.ds(j * 16, 16)] = zeros
            return carry

        lax.fori_loop(0, NP // 16, zbody, 0)
        ones = jnp.ones((16,), jnp.float32)

        def body(j, carry):
            idx16 = idx_d[pl.ds(j * 16, 16)]
            plsc.addupdate_scatter(deg_v, [idx16], ones)
            return carry

        lax.fori_loop(0, EPT // 16, body, 0, unroll=8)
        pltpu.sync_copy(deg_v, out_hbm.at[wid])

    return k(dst)


def _sc_pass(d, bb, pc, nph, ring, xwp, src3, dst3, zeros_blk):
    G = ring - 1 if ring > 2 else 1

    @functools.partial(
        pl.kernel,
        out_type=jax.ShapeDtypeStruct((2, NP, d), jnp.float32),
        mesh=_sc_mesh(),
        compiler_params=pltpu.CompilerParams(
            use_tc_tiling_on_sc=(d % 128 == 0)),
        scratch_types=[
            pltpu.VMEM_SHARED((NP, d), jnp.float32),
            pltpu.VMEM((2, pc, bb), jnp.int32),
            pltpu.VMEM((2, pc, bb), jnp.int32),
            pltpu.VMEM((ring, bb, d), jnp.float32),
            [pltpu.SemaphoreType.DMA] * ring,
            [pltpu.SemaphoreType.DMA] * ring,
            pltpu.SemaphoreType.DMA,
        ],
    )
    def k(xwp_hbm, src_hbm, dst_hbm, zeros_hbm, out_hbm,
          acc_sh, src_v, dst_v, rows_v, sg, ss, sem_i):
        c = lax.axis_index("c")
        s = lax.axis_index("s")
        wid = c * 16 + s
        r0 = s * RPT
        nrows = xwp_hbm.shape[0]

        pltpu.async_copy(src_hbm.at[wid * nph], src_v.at[0], sem_i)
        pltpu.async_copy(dst_hbm.at[wid * nph], dst_v.at[0], sem_i)

        if nrows == NP:
            @pl.when(c == 0)
            def _():
                pltpu.sync_copy(xwp_hbm.at[pl.ds(r0, RPT), :],
                                acc_sh.at[pl.ds(r0, RPT), :])
        else:
            tail = N - 15 * RPT
            @pl.when(jnp.logical_and(c == 0, s < 15))
            def _():
                pltpu.sync_copy(xwp_hbm.at[pl.ds(r0, RPT), :],
                                acc_sh.at[pl.ds(r0, RPT), :])

            @pl.when(jnp.logical_and(c == 0, s == 15))
            def _():
                pltpu.sync_copy(xwp_hbm.at[pl.ds(15 * RPT, tail), :],
                                acc_sh.at[pl.ds(15 * RPT, tail), :])
                pltpu.sync_copy(zeros_hbm.at[pl.ds(0, NP - N), :],
                                acc_sh.at[pl.ds(N, NP - N), :])

        @pl.when(c != 0)
        def _():
            pltpu.sync_copy(zeros_hbm, acc_sh.at[pl.ds(r0, RPT), :])

        plsc.subcore_barrier()

        for p in range(nph):
            pb = p % 2
            pltpu.make_async_copy(src_hbm.at[wid * nph + p],
                                  src_v.at[pb], sem_i).wait()
            pltpu.make_async_copy(dst_hbm.at[wid * nph + p],
                                  dst_v.at[pb], sem_i).wait()
            if p + 1 < nph:
                pltpu.async_copy(src_hbm.at[wid * nph + p + 1],
                                 src_v.at[1 - pb], sem_i)
                pltpu.async_copy(dst_hbm.at[wid * nph + p + 1],
                                 dst_v.at[1 - pb], sem_i)

            for g in range(G):
                pltpu.async_copy(xwp_hbm.at[src_v.at[pb, g]],
                                 rows_v.at[g], sg[g])

            def body(jr, carry):
                jbase = jr * ring
                for u in range(ring):
                    j = jbase + u
                    nu = (u + G) % ring
                    pltpu.make_async_copy(xwp_hbm.at[src_v.at[pb, j]],
                                          rows_v.at[u], sg[u]).wait()
                    pltpu.async_copy(rows_v.at[u],
                                     acc_sh.at[dst_v.at[pb, j]],
                                     ss[u], add=True)
                    if u < ring - G:
                        @pl.when(jr > 0)
                        def _(j=j, u=u, nu=nu):
                            pltpu.make_async_copy(
                                rows_v.at[nu],
                                acc_sh.at[dst_v.at[pb, j + G - ring]],
                                ss[nu]).wait()
                    else:
                        pltpu.make_async_copy(
                            rows_v.at[nu],
                            acc_sh.at[dst_v.at[pb, j + G - ring]],
                            ss[nu]).wait()

                    @pl.when(j + G < pc)
                    def _(j=j, u=u, nu=nu):
                        pltpu.async_copy(xwp_hbm.at[src_v.at[pb, j + G]],
                                         rows_v.at[nu], sg[nu])
                return carry

            lax.fori_loop(0, pc // ring, body, 0)
            for j in range(pc - (ring - G), pc):
                pltpu.make_async_copy(rows_v.at[j % ring],
                                      acc_sh.at[dst_v.at[pb, j]],
                                      ss[j % ring]).wait()

        plsc.subcore_barrier()
        pltpu.sync_copy(acc_sh.at[pl.ds(r0, RPT), :],
                        out_hbm.at[c, pl.ds(r0, RPT), :])

    return k(xwp, src3, dst3, zeros_blk)


def _tc_mm1(x, W1, degp):
    def body(x_ref, w_ref, deg_ref, xw_ref, dinv_ref):
        degt = jnp.transpose(deg_ref[...])
        deg = jnp.sum(degt, axis=1, keepdims=True) + 1.0
        dinv = lax.rsqrt(deg)
        dinv_ref[...] = jnp.broadcast_to(dinv, (ROWBLK, 16))
        xw = jnp.dot(x_ref[...], w_ref[...],
                     preferred_element_type=jnp.float32)
        xw_ref[...] = xw * dinv

    return pl.pallas_call(
        body,
        grid=(GRID,),
        in_specs=[
            pl.BlockSpec((ROWBLK, 128), lambda i: (i, 0)),
            pl.BlockSpec((128, 128), lambda i: (0, 0)),
            pl.BlockSpec((NTILES, ROWBLK), lambda i: (0, i)),
        ],
        out_specs=[
            pl.BlockSpec((ROWBLK, 128), lambda i: (i, 0)),
            pl.BlockSpec((ROWBLK, 16), lambda i: (i, 0)),
        ],
        out_shape=[
            jax.ShapeDtypeStruct((NP, 128), jnp.float32),
            jax.ShapeDtypeStruct((NP, 16), jnp.float32),
        ],
    )(x, W1, degp)


def _tc_mm2(acc1, dinvw, b1, W2):
    def body(a_ref, dinv_ref, b_ref, w_ref, out_ref):
        dinv = dinv_ref[...][:, 0:1]
        pre = (a_ref[0] + a_ref[1]) * dinv + b_ref[...]
        h = jnp.maximum(pre, 0.0)
        xw = jnp.dot(h, w_ref[...], preferred_element_type=jnp.float32)
        out_ref[...] = xw * dinv

    return pl.pallas_call(
        body,
        grid=(N // NRB,),
        in_specs=[
            pl.BlockSpec((2, NRB, 128), lambda i: (0, i, 0)),
            pl.BlockSpec((NRB, 16), lambda i: (i, 0)),
            pl.BlockSpec((1, 128), lambda i: (0, 0)),
            pl.BlockSpec((128, 64), lambda i: (0, 0)),
        ],
        out_specs=pl.BlockSpec((NRB, 64), lambda i: (i, 0)),
        out_shape=jax.ShapeDtypeStruct((N, 64), jnp.float32),
    )(acc1, dinvw, b1, W2)


def _tc_final(acc2, dinvw, b2, clusterT):
    def body(a_ref, dinv_ref, b_ref, ct_ref, feat_ref, q_ref):
        dinv = dinv_ref[...][:, 0:1]
        f = (a_ref[0] + a_ref[1]) * dinv + b_ref[...]
        feat_ref[...] = f
        ct = ct_ref[...]
        fsq = jnp.sum(f * f, axis=1, keepdims=True)
        csq = jnp.sum(ct * ct, axis=0, keepdims=True)
        fc = jnp.dot(f, ct, preferred_element_type=jnp.float32)
        dist = fsq - 2.0 * fc + csq
        u = 1.0 + dist * 0.5
        r = lax.rsqrt(u)
        qun = r * r * r
        q_ref[...] = qun / jnp.sum(qun, axis=1, keepdims=True)

    return pl.pallas_call(
        body,
        grid=(N // NRB,),
        in_specs=[
            pl.BlockSpec((2, NRB, 64), lambda i: (0, i, 0)),
            pl.BlockSpec((NRB, 16), lambda i: (i, 0)),
            pl.BlockSpec((1, 64), lambda i: (0, 0)),
            pl.BlockSpec((64, 16), lambda i: (0, 0)),
        ],
        out_specs=[
            pl.BlockSpec((NRB, 64), lambda i: (i, 0)),
            pl.BlockSpec((NRB, 16), lambda i: (i, 0)),
        ],
        out_shape=[
            jax.ShapeDtypeStruct((N, 64), jnp.float32),
            jax.ShapeDtypeStruct((N, 16), jnp.float32),
        ],
    )(acc2, dinvw, b2, clusterT)


def kernel(x, edge_index, W1, b1, W2, b2, cluster):
    src = edge_index[0].astype(jnp.int32)
    dst = edge_index[1].astype(jnp.int32)
    x_p = jnp.zeros((NP, 128), jnp.float32).at[:N].set(x)

    zeros128 = jnp.zeros((RPT, 128), jnp.float32)
    zeros64 = jnp.zeros((RPT, 64), jnp.float32)

    src3a = src.reshape(NTILES * NPH1, PC1, B1)
    dst3a = dst.reshape(NTILES * NPH1, PC1, B1)
    src3b = src.reshape(NTILES * NPH2, PC2, B2)
    dst3b = dst.reshape(NTILES * NPH2, PC2, B2)
    degp = _sc_degree(dst)
    xw1p, dinvw = _tc_mm1(x_p, W1, degp)
    acc1 = _sc_pass(128, B1, PC1, NPH1, R1, xw1p, src3a, dst3a, zeros128)
    xw2p = _tc_mm2(acc1, dinvw, b1.reshape(1, 128), W2)
    acc2 = _sc_pass(64, B2, PC2, NPH2, R2, xw2p, src3b, dst3b, zeros64)
    feature, q = _tc_final(acc2, dinvw, b2.reshape(1, 64), cluster.T)
    return (feature, q)
